# baseline (speedup 1.0000x reference)
import functools

import jax
import jax.numpy as jnp
import numpy as np
from jax import lax
from jax.experimental import pallas as pl
from jax.experimental.pallas import tpu as pltpu
from jax.experimental.pallas import tpu_sc as plsc

F = 26
E = 16
A = 16
B = 4096
FIELD_DIM = 100000
TOTAL = F * FIELD_DIM
_OFFSETS = np.arange(F, dtype=np.int32) * FIELD_DIM

NC, NS = 2, 16
NW = NC * NS
BF = B * F
PER_W = BF // NW
LANES = 128
CHUNKS = PER_W // LANES
RPL = 128 // E
LIN_ROWS = (TOTAL + 127) // 128


def _gather_body(tblr, linr, idx8_hbm, rem8_hbm, idxl_hbm, reml_hbm,
                 emb_out, lin_out, idx8_v, rem8_v, idxl_v, reml_v,
                 stage_v, tmp_v, lv_v, sem):
    wid = lax.axis_index("s") * NC + lax.axis_index("c")
    pltpu.sync_copy(idx8_hbm.at[wid], idx8_v)
    pltpu.sync_copy(rem8_hbm.at[wid], rem8_v)
    pltpu.sync_copy(idxl_hbm.at[wid], idxl_v)
    pltpu.sync_copy(reml_hbm.at[wid], reml_v)
    iota16 = lax.broadcasted_iota(jnp.int32, (16,), 0)

    def emb_chunk(j, carry):
        pltpu.async_copy(tblr.at[idx8_v.at[j]], stage_v, sem).wait()

        def group(g, c2):
            rows16 = iota16 + g * 16
            rem16 = rem8_v[j, pl.ds(g * 16, 16)]
            lanebase = rem16 * 16
            for e in range(E):
                v = plsc.load_gather(stage_v, [rows16, lanebase + e])
                plsc.store_scatter(tmp_v, [rows16, jnp.full((16,), e, jnp.int32)], v)
            return c2

        lax.fori_loop(0, LANES // 16, group, 0, unroll=False)
        pltpu.sync_copy(tmp_v, emb_out.at[wid, j])
        return carry

    lax.fori_loop(0, CHUNKS, emb_chunk, 0, unroll=False)

    def lin_chunk(j, carry):
        pltpu.async_copy(linr.at[idxl_v.at[j]], stage_v, sem).wait()

        def group(g, c2):
            rows16 = iota16 + g * 16
            rem16 = reml_v[j, pl.ds(g * 16, 16)]
            v = plsc.load_gather(stage_v, [rows16, rem16])
            lv_v[j, pl.ds(g * 16, 16)] = v
            return c2

        lax.fori_loop(0, LANES // 16, group, 0, unroll=False)
        return carry

    lax.fori_loop(0, CHUNKS, lin_chunk, 0, unroll=False)
    pltpu.sync_copy(lv_v, lin_out.at[wid])


def _sc_gather(tableR, linR, idx8, rem8, idxl, reml):
    mesh = plsc.VectorSubcoreMesh(core_axis_name="c", subcore_axis_name="s",
                                  num_cores=NC, num_subcores=NS)
    run = functools.partial(
        pl.kernel,
        out_type=[
            jax.ShapeDtypeStruct((NW, CHUNKS, LANES, E), jnp.float32),
            jax.ShapeDtypeStruct((NW, CHUNKS, LANES), jnp.float32),
        ],
        mesh=mesh,
        scratch_types=[
            pltpu.VMEM((CHUNKS, LANES), jnp.int32),
            pltpu.VMEM((CHUNKS, LANES), jnp.int32),
            pltpu.VMEM((CHUNKS, LANES), jnp.int32),
            pltpu.VMEM((CHUNKS, LANES), jnp.int32),
            pltpu.VMEM((LANES, 128), jnp.float32),
            pltpu.VMEM((LANES, E), jnp.float32),
            pltpu.VMEM((CHUNKS, LANES), jnp.float32),
            pltpu.SemaphoreType.DMA,
        ],
        compiler_params=pltpu.CompilerParams(needs_layout_passes=False),
    )(_gather_body)
    return run(tableR, linR, idx8, rem8, idxl, reml)


BT = 16


def _afm_body(emb_ref, linv_ref, attn_w_ref, attn_b_ref, projt_ref,
              proj_b_ref, fct_ref, fc_b_ref, bias_ref, out_ref):
    e3 = emb_ref[...]
    inner = e3[:, :, None, :] * e3[:, None, :, :]
    innf = inner.reshape(BT * F * F, E)
    att = jnp.dot(innf.astype(jnp.bfloat16),
                  attn_w_ref[...].astype(jnp.bfloat16),
                  preferred_element_type=jnp.float32)
    att = jnp.maximum(att + attn_b_ref[...], 0.0)
    logits = jnp.sum(att.reshape(BT, F, F, A) * projt_ref[...][None, None],
                     axis=-1) + proj_b_ref[0, 0]
    ii = lax.broadcasted_iota(jnp.int32, (F, F), 0)
    jj = lax.broadcasted_iota(jnp.int32, (F, F), 1)
    mask = (jj > ii)[None]
    logits = jnp.where(mask, logits, -1e30)
    m = jnp.max(jnp.max(logits, axis=2), axis=1)
    ex = jnp.where(mask, jnp.exp(logits - m[:, None, None]), 0.0)
    s = jnp.sum(jnp.sum(ex, axis=2), axis=1)
    scores = ex / s[:, None, None]
    afm = jnp.sum(jnp.sum(scores[..., None] * inner, axis=2), axis=1)
    lin = jnp.sum(linv_ref[...], axis=1, keepdims=True)
    out = lin + bias_ref[0, 0] + fc_b_ref[0, 0] + jnp.sum(
        afm * fct_ref[...], axis=1, keepdims=True)
    out_ref[...] = out


def _afm_tc(emb3, linv, attn_W, attn_b, proj_W, proj_b, fc_W, fc_b, bias):
    rep = lambda i: (0, 0)
    return pl.pallas_call(
        _afm_body,
        grid=(B // BT,),
        in_specs=[
            pl.BlockSpec((BT, F, E), lambda i: (i, 0, 0)),
            pl.BlockSpec((BT, F), lambda i: (i, 0)),
            pl.BlockSpec((E, A), rep),
            pl.BlockSpec((1, A), rep),
            pl.BlockSpec((1, A), rep),
            pl.BlockSpec((1, 1), rep),
            pl.BlockSpec((1, E), rep),
            pl.BlockSpec((1, 1), rep),
            pl.BlockSpec((1, 1), rep),
        ],
        out_specs=pl.BlockSpec((BT, 1), lambda i: (i, 0)),
        out_shape=jax.ShapeDtypeStruct((B, 1), jnp.float32),
    )(emb3, linv, attn_W, attn_b.reshape(1, A), proj_W.reshape(1, A),
      proj_b.reshape(1, 1), fc_W.reshape(1, E), fc_b.reshape(1, 1),
      bias.reshape(1, 1))


def kernel(x, table, linear_w, bias, attn_W, attn_b, proj_W, proj_b, fc_W, fc_b):
    tableR = table.reshape(TOTAL // RPL, 128)
    lin_flat = jnp.concatenate(
        [linear_w[:, 0], jnp.zeros((LIN_ROWS * 128 - TOTAL,), jnp.float32)])
    linR = lin_flat.reshape(LIN_ROWS, 128)
    idx = x + jnp.asarray(_OFFSETS)[None, :]
    idx8 = (idx >> 3).reshape(NW, CHUNKS, LANES)
    rem8 = (idx & 7).reshape(NW, CHUNKS, LANES)
    idxl = (idx >> 7).reshape(NW, CHUNKS, LANES)
    reml = (idx & 127).reshape(NW, CHUNKS, LANES)
    emb4, lin3 = _sc_gather(tableR, linR, idx8, rem8, idxl, reml)
    emb3 = emb4.reshape(B, F, E)
    linv = lin3.reshape(B, F)
    return _afm_tc(emb3, linv, attn_W, attn_b, proj_W, proj_b, fc_W, fc_b, bias)


# speedup vs baseline: 2.1044x; 2.1044x over previous
import functools

import jax
import jax.numpy as jnp
import numpy as np
from jax import lax
from jax.experimental import pallas as pl
from jax.experimental.pallas import tpu as pltpu
from jax.experimental.pallas import tpu_sc as plsc

F = 26
E = 16
A = 16
B = 4096
FIELD_DIM = 100000
TOTAL = F * FIELD_DIM
_OFFSETS = np.arange(F, dtype=np.int32) * FIELD_DIM

NC, NS = 2, 16
NW = NC * NS
BF = B * F
PER_W = BF // NW
LANES = 128
CHUNKS = PER_W // LANES
RPL = 128 // E
LIN_ROWS = (TOTAL + 127) // 128


def _gather_body(tblr, linr, idx8_hbm, rem8_hbm, idxl_hbm, reml_hbm,
                 emb_out, lin_out, idx8_v, rem8_v, idxl_v, reml_v,
                 stage_v, tmp_v, lv_v, sem):
    wid = lax.axis_index("s") * NC + lax.axis_index("c")
    pltpu.sync_copy(idx8_hbm.at[wid], idx8_v)
    pltpu.sync_copy(rem8_hbm.at[wid], rem8_v)
    pltpu.sync_copy(idxl_hbm.at[wid], idxl_v)
    pltpu.sync_copy(reml_hbm.at[wid], reml_v)
    iota16 = lax.broadcasted_iota(jnp.int32, (16,), 0)

    def emb_chunk(j, carry):
        pltpu.async_copy(tblr.at[idx8_v.at[j]], stage_v, sem).wait()

        def group(g, c2):
            rows16 = iota16 + g * 16
            rem16 = rem8_v[j, pl.ds(g * 16, 16)]
            lanebase = rem16 * 16
            for e in range(E):
                v = plsc.load_gather(stage_v, [rows16, lanebase + e])
                plsc.store_scatter(tmp_v, [rows16, jnp.full((16,), e, jnp.int32)], v)
            return c2

        lax.fori_loop(0, LANES // 16, group, 0, unroll=False)
        pltpu.sync_copy(tmp_v, emb_out.at[wid, j])
        return carry

    lax.fori_loop(0, CHUNKS, emb_chunk, 0, unroll=False)

    def lin_chunk(j, carry):
        pltpu.async_copy(linr.at[idxl_v.at[j]], stage_v, sem).wait()

        def group(g, c2):
            rows16 = iota16 + g * 16
            rem16 = reml_v[j, pl.ds(g * 16, 16)]
            v = plsc.load_gather(stage_v, [rows16, rem16])
            lv_v[j, pl.ds(g * 16, 16)] = v
            return c2

        lax.fori_loop(0, LANES // 16, group, 0, unroll=False)
        return carry

    lax.fori_loop(0, CHUNKS, lin_chunk, 0, unroll=False)
    pltpu.sync_copy(lv_v, lin_out.at[wid])


def _sc_gather(tableR, linR, idx8, rem8, idxl, reml):
    mesh = plsc.VectorSubcoreMesh(core_axis_name="c", subcore_axis_name="s",
                                  num_cores=NC, num_subcores=NS)
    run = functools.partial(
        pl.kernel,
        out_type=[
            jax.ShapeDtypeStruct((NW, CHUNKS, LANES, E), jnp.float32),
            jax.ShapeDtypeStruct((NW, CHUNKS, LANES), jnp.float32),
        ],
        mesh=mesh,
        scratch_types=[
            pltpu.VMEM((CHUNKS, LANES), jnp.int32),
            pltpu.VMEM((CHUNKS, LANES), jnp.int32),
            pltpu.VMEM((CHUNKS, LANES), jnp.int32),
            pltpu.VMEM((CHUNKS, LANES), jnp.int32),
            pltpu.VMEM((LANES, 128), jnp.float32),
            pltpu.VMEM((LANES, E), jnp.float32),
            pltpu.VMEM((CHUNKS, LANES), jnp.float32),
            pltpu.SemaphoreType.DMA,
        ],
        compiler_params=pltpu.CompilerParams(needs_layout_passes=False),
    )(_gather_body)
    return run(tableR, linR, idx8, rem8, idxl, reml)


BT = 128
FP = 32
W = FP * E


def _afm_body(embp_ref, embr_ref, linv_ref, t16_ref, w32_ref, ab_ref,
              pv_ref, t32_ref, fct_ref, consts_ref, out_ref):
    bf = jnp.bfloat16
    xp = embp_ref[...].astype(bf)
    r2 = jnp.dot(embr_ref[...].astype(bf), t16_ref[...],
                 preferred_element_type=jnp.float32).astype(bf)
    inner = r2.reshape(BT, F, W) * xp[:, None, :]
    inner2 = inner.reshape(BT * F, W)
    att = jnp.dot(inner2, w32_ref[...], preferred_element_type=jnp.float32)
    att = jnp.maximum(att + ab_ref[...], 0.0)
    logits = jnp.dot(att.astype(bf), pv_ref[...],
                     preferred_element_type=jnp.float32)
    logits = logits + consts_ref[0, 0]
    lg3 = logits.reshape(BT, F, FP)
    ii = lax.broadcasted_iota(jnp.int32, (F, FP), 0)
    jj = lax.broadcasted_iota(jnp.int32, (F, FP), 1)
    mask = ((jj > ii) & (jj < F))[None]
    lg3 = jnp.where(mask, lg3, -1e30)
    m = jnp.max(jnp.max(lg3, axis=2), axis=1)
    ex = jnp.where(mask, jnp.exp(lg3 - m[:, None, None]), 0.0)
    s = jnp.sum(jnp.sum(ex, axis=2), axis=1)
    scores = (ex / s[:, None, None]).reshape(BT * F, FP)
    srep = jnp.dot(scores.astype(bf), t32_ref[...],
                   preferred_element_type=jnp.float32).astype(bf)
    ws = (srep * inner2).astype(jnp.float32).reshape(BT, F, W)
    sums = jnp.sum(ws, axis=1)
    afm = jnp.dot(sums.astype(bf), fct_ref[...].astype(bf),
                  preferred_element_type=jnp.float32)
    lin = jnp.sum(linv_ref[...], axis=1, keepdims=True)
    out_ref[...] = lin + afm + consts_ref[0, 1]


def _afm_tc(embp, embr, linv, t16, w32, abt, pv, t32, fct, consts):
    rep = lambda i: (0, 0)
    return pl.pallas_call(
        _afm_body,
        grid=(B // BT,),
        in_specs=[
            pl.BlockSpec((BT, W), lambda i: (i, 0)),
            pl.BlockSpec((BT * F, E), lambda i: (i, 0)),
            pl.BlockSpec((BT, F), lambda i: (i, 0)),
            pl.BlockSpec((E, W), rep),
            pl.BlockSpec((W, W), rep),
            pl.BlockSpec((1, W), rep),
            pl.BlockSpec((W, FP), rep),
            pl.BlockSpec((FP, W), rep),
            pl.BlockSpec((W, 1), rep),
            pl.BlockSpec((1, 2), rep),
        ],
        out_specs=pl.BlockSpec((BT, 1), lambda i: (i, 0)),
        out_shape=jax.ShapeDtypeStruct((B, 1), jnp.float32),
    )(embp, embr, linv, t16, w32, abt, pv, t32, fct, consts)


def kernel(x, table, linear_w, bias, attn_W, attn_b, proj_W, proj_b, fc_W, fc_b):
    tableR = table.reshape(TOTAL // RPL, 128)
    lin_flat = jnp.concatenate(
        [linear_w[:, 0], jnp.zeros((LIN_ROWS * 128 - TOTAL,), jnp.float32)])
    linR = lin_flat.reshape(LIN_ROWS, 128)
    idx = x + jnp.asarray(_OFFSETS)[None, :]
    idx8 = (idx >> 3).reshape(NW, CHUNKS, LANES)
    rem8 = (idx & 7).reshape(NW, CHUNKS, LANES)
    idxl = (idx >> 7).reshape(NW, CHUNKS, LANES)
    reml = (idx & 127).reshape(NW, CHUNKS, LANES)
    emb4, lin3 = _sc_gather(tableR, linR, idx8, rem8, idxl, reml)
    embr = emb4.reshape(B * F, E)
    embp = jnp.pad(emb4.reshape(B, F * E), ((0, 0), (0, (FP - F) * E)))
    linv = lin3.reshape(B, F)
    bf = jnp.bfloat16
    eye32 = jnp.eye(FP, dtype=jnp.float32)
    t16 = jnp.tile(jnp.eye(E, dtype=jnp.float32), (1, FP)).astype(bf)
    w32 = jnp.kron(eye32, attn_W).astype(bf)
    abt = jnp.tile(attn_b, FP).reshape(1, W)
    pv = jnp.kron(eye32, proj_W).astype(bf)
    t32 = jnp.kron(eye32, jnp.ones((1, E), jnp.float32)).astype(bf)
    fmask = (jnp.arange(FP) < F).astype(jnp.float32).reshape(FP, 1)
    fct = jnp.kron(fmask, fc_W)
    consts = jnp.stack([proj_b[0], bias[0] + fc_b[0]]).reshape(1, 2)
    return _afm_tc(embp, embr, linv, t16, w32, abt, pv, t32, fct, consts)
